# Initial kernel scaffold; baseline (speedup 1.0000x reference)
import jax
import jax.numpy as jnp
from jax.experimental import pallas as pl


def kernel(z_flat, embedding):
    raise NotImplementedError("write your pallas kernel here")


# speedup vs baseline: 1.4625x; 1.4625x over previous
import functools

import jax
import jax.numpy as jnp
from jax import lax
from jax.experimental import pallas as pl
from jax.experimental.pallas import tpu as pltpu
from jax.experimental.pallas import tpu_sc as plsc

N_TOK = 16384
K = 1024
D = 256
ROWS = 1024
GRID = N_TOK // ROWS

NW = 32
B_PER_W = N_TOK // NW
CH = 4
ROWS_CH = B_PER_W // CH


def _dist_argmin_body(z_ref, e_ref, idx_ref, minsum_ref):
    i = pl.program_id(0)
    z = z_ref[...]
    e = e_ref[...]
    zsq = jnp.sum(z * z, axis=1, keepdims=True)
    esq = jnp.sum(e * e, axis=1)
    prod = lax.dot_general(z, e, (((1,), (1,)), ((), ())),
                           preferred_element_type=jnp.float32)
    dist = (zsq + esq[None, :]) - 2.0 * prod
    minval = jnp.min(dist, axis=1)
    iota = lax.broadcasted_iota(jnp.int32, dist.shape, 1)
    idx = jnp.min(jnp.where(dist == minval[:, None], iota, jnp.int32(K)),
                  axis=1)
    idx_ref[0, 0, :] = idx

    @pl.when(i == 0)
    def _():
        minsum_ref[0, 0] = 0.0

    minsum_ref[0, 0] += jnp.sum(minval)


def _dist_argmin(z_flat, embedding):
    return pl.pallas_call(
        _dist_argmin_body,
        grid=(GRID,),
        in_specs=[
            pl.BlockSpec((ROWS, D), lambda i: (i, 0)),
            pl.BlockSpec((K, D), lambda i: (0, 0)),
        ],
        out_specs=[
            pl.BlockSpec((1, 1, ROWS), lambda i: (i, 0, 0)),
            pl.BlockSpec(memory_space=pltpu.SMEM),
        ],
        out_shape=[
            jax.ShapeDtypeStruct((GRID, 1, ROWS), jnp.int32),
            jax.ShapeDtypeStruct((1, 1), jnp.float32),
        ],
    )(z_flat, embedding)


def _sc_gather(embedding, idx_grouped):
    mesh = plsc.VectorSubcoreMesh(core_axis_name="c", subcore_axis_name="s")

    @functools.partial(
        pl.kernel,
        mesh=mesh,
        out_type=jax.ShapeDtypeStruct((N_TOK, D), jnp.float32),
        scratch_types=[
            pltpu.VMEM((CH, ROWS_CH), jnp.int32),
            pltpu.VMEM((ROWS_CH, D), jnp.float32),
            pltpu.SemaphoreType.DMA,
        ],
    )
    def gather_k(table_hbm, idx_hbm, out_hbm, idx_v, rows_v, sem):
        wid = lax.axis_index("s") * 2 + lax.axis_index("c")
        base = wid * B_PER_W
        pltpu.sync_copy(idx_hbm.at[wid], idx_v)
        for c in range(CH):
            pltpu.async_copy(table_hbm.at[idx_v.at[c]], rows_v, sem).wait()
            pltpu.sync_copy(rows_v, out_hbm.at[pl.ds(base + c * ROWS_CH, ROWS_CH)])

    return gather_k(embedding, idx_grouped)


def kernel(z_flat, embedding):
    idx3, minsum = _dist_argmin(z_flat, embedding)
    indices = idx3.reshape(N_TOK)
    loss = minsum[0, 0] * (1.25 / (N_TOK * D))
    quantized = _sc_gather(embedding, indices.reshape(NW, CH, ROWS_CH))
    return (loss, quantized, indices)
